# Initial kernel scaffold; baseline (speedup 1.0000x reference)
import jax
import jax.numpy as jnp
from jax.experimental import pallas as pl


def kernel(x, W1_self, W1_neigh, b1, W2_self, W2_neigh, b2, Wc1, bc1, Wc2, bc2, edge_index):
    raise NotImplementedError("write your pallas kernel here")


# speedup vs baseline: 4.8171x; 4.8171x over previous
import functools

import jax
import jax.numpy as jnp
from jax import lax
from jax.experimental import pallas as pl
from jax.experimental.pallas import tpu as pltpu
from jax.experimental.pallas import tpu_sc as plsc

N = 10000
D = 128
C = 16
E = 320000

NC = 2
NS = 16
CH = 128
E_PAD = 327680
CHUNKS = E_PAD // (NS * CH)
ROWS_PER_TILE = N // NS
DE1 = 160
W1C = DE1 // NC
W2C = D // NC
BN = 2000
GRID = N // BN


def _make_sc_agg(W):
    mesh = plsc.VectorSubcoreMesh(
        core_axis_name="c", subcore_axis_name="s",
        num_cores=NC, num_subcores=NS)

    @functools.partial(
        pl.kernel,
        out_type=jax.ShapeDtypeStruct((NC, N, W), jnp.float32),
        mesh=mesh,
        scratch_types=[
            pltpu.VMEM((CHUNKS, CH), jnp.int32),
            pltpu.VMEM((CHUNKS, CH), jnp.int32),
            pltpu.VMEM((CH, W), jnp.float32),
            pltpu.VMEM((CH, W), jnp.float32),
            pltpu.VMEM_SHARED((N, W), jnp.float32),
            pltpu.SemaphoreType.DMA,
            pltpu.SemaphoreType.DMA,
        ],
        compiler_params=pltpu.CompilerParams(use_tc_tiling_on_sc=False),
    )
    def sc_agg(feats, srcs, dsts, out, src_v, dst_v, buf0, buf1, acc, sem0, sem1):
        c = lax.axis_index("c")
        s = lax.axis_index("s")
        bufs = (buf0, buf1)
        sems = (sem0, sem1)
        feat = feats.at[c]

        zero16 = jnp.zeros((16,), jnp.float32)

        def zrow(r, carry):
            for k in range(W // 16):
                buf0[r, pl.ds(k * 16, 16)] = zero16
            return carry
        lax.fori_loop(0, CH, zrow, 0)
        base = s * ROWS_PER_TILE
        full = ROWS_PER_TILE // CH
        for t in range(full):
            pltpu.sync_copy(buf0, acc.at[pl.ds(base + t * CH, CH)])
        rem = ROWS_PER_TILE - full * CH
        pltpu.sync_copy(buf0.at[pl.ds(0, rem)],
                        acc.at[pl.ds(base + full * CH, rem)])
        plsc.subcore_barrier()

        pltpu.sync_copy(srcs.at[s], src_v)
        pltpu.sync_copy(dsts.at[s], dst_v)

        pltpu.async_copy(feat.at[src_v.at[0]], buf0, sem0)
        pltpu.async_copy(feat.at[src_v.at[1]], buf1, sem1)

        def body(i, carry):
            for b in range(2):
                j = i * 2 + b
                pltpu.make_async_copy(feat.at[src_v.at[j]], bufs[b], sems[b]).wait()
                pltpu.sync_copy(bufs[b], acc.at[dst_v.at[j]], add=True)
                pltpu.async_copy(feat.at[src_v.at[j + 2]], bufs[b], sems[b])
            return carry
        lax.fori_loop(0, CHUNKS // 2 - 1, body, 0)
        for b in range(2):
            j = CHUNKS - 2 + b
            pltpu.make_async_copy(feat.at[src_v.at[j]], bufs[b], sems[b]).wait()
            pltpu.sync_copy(bufs[b], acc.at[dst_v.at[j]], add=True)

        plsc.subcore_barrier()
        pltpu.sync_copy(acc.at[pl.ds(base, ROWS_PER_TILE)],
                        out.at[c, pl.ds(base, ROWS_PER_TILE)])

    return sc_agg


_sc_agg1 = _make_sc_agg(W1C)
_sc_agg2 = _make_sc_agg(W2C)


def _tc_layer1(x_ref, p0, p1, ws, wn, b, o, oinv):
    pa = p0[...]
    pb = p1[...]
    deg = jnp.sum(pb[:, D - W1C:], axis=1, keepdims=True)
    inv = 1.0 / jnp.maximum(deg, 1.0)
    agg = jnp.concatenate([pa, pb[:, :D - W1C]], axis=1) * inv
    h = x_ref[...] @ ws[...] + agg @ wn[...] + b[...]
    o[...] = jnp.maximum(h, 0.0)
    oinv[...] = jnp.broadcast_to(inv, (BN, 8))


def _tc_layer2(h_ref, q0, q1, dinv, w2s, w2n, b2, wc1, bc1, wc2, bc2, o):
    inv = dinv[...][:, 0:1]
    agg = jnp.concatenate([q0[...], q1[...]], axis=1) * inv
    h2 = jnp.maximum(h_ref[...] @ w2s[...] + agg @ w2n[...] + b2[...], 0.0)
    h3 = jnp.maximum(h2 @ wc1[...] + bc1[...], 0.0)
    o[...] = h3 @ wc2[...] + bc2[...]


def _row_spec(w):
    return pl.BlockSpec((BN, w), lambda i: (i, 0))


def _full_spec(h, w):
    return pl.BlockSpec((h, w), lambda i: (0, 0))


_tck1 = pl.pallas_call(
    _tc_layer1,
    grid=(GRID,),
    in_specs=[_row_spec(D), _row_spec(W1C), _row_spec(W1C),
              _full_spec(D, D), _full_spec(D, D), _full_spec(1, D)],
    out_specs=[_row_spec(D), pl.BlockSpec((BN, 8), lambda i: (i, 0))],
    out_shape=[jax.ShapeDtypeStruct((N, D), jnp.float32),
               jax.ShapeDtypeStruct((N, 8), jnp.float32)],
)

_tck2 = pl.pallas_call(
    _tc_layer2,
    grid=(GRID,),
    in_specs=[_row_spec(D), _row_spec(W2C), _row_spec(W2C),
              pl.BlockSpec((BN, 8), lambda i: (i, 0)),
              _full_spec(D, D), _full_spec(D, D), _full_spec(1, D),
              _full_spec(D, D), _full_spec(1, D),
              _full_spec(D, C), _full_spec(1, C)],
    out_specs=pl.BlockSpec((BN, C), lambda i: (i, 0)),
    out_shape=jax.ShapeDtypeStruct((N, C), jnp.float32),
)


def kernel(x, W1_self, W1_neigh, b1, W2_self, W2_neigh, b2,
           Wc1, bc1, Wc2, bc2, edge_index):
    zrow = jnp.zeros((1, W1C), jnp.float32)
    fa = jnp.concatenate([x[:, :W1C], zrow], axis=0)
    fb = jnp.concatenate([
        jnp.concatenate([x[:, W1C:], jnp.ones((N, 1), jnp.float32),
                         jnp.zeros((N, DE1 - D - 1), jnp.float32)], axis=1),
        zrow], axis=0)
    feats1 = jnp.stack([fa, fb])

    pad = E_PAD - E
    src = jnp.concatenate([edge_index[0], jnp.full((pad,), N, jnp.int32)])
    dst = jnp.concatenate([edge_index[1], jnp.zeros((pad,), jnp.int32)])
    src = src.reshape(NS, CHUNKS, CH)
    dst = dst.reshape(NS, CHUNKS, CH)

    p1 = _sc_agg1(feats1, src, dst)
    h1, inv_deg = _tck1(x, p1[0], p1[1], W1_self, W1_neigh, b1.reshape(1, D))

    zrow2 = jnp.zeros((1, W2C), jnp.float32)
    feats2 = jnp.stack([
        jnp.concatenate([h1[:, :W2C], zrow2], axis=0),
        jnp.concatenate([h1[:, W2C:], zrow2], axis=0)])
    p2 = _sc_agg2(feats2, src, dst)

    out = _tck2(h1, p2[0], p2[1], inv_deg,
                W2_self, W2_neigh, b2.reshape(1, D),
                Wc1, bc1.reshape(1, D), Wc2, bc2.reshape(1, C))
    return out


# speedup vs baseline: 4.8792x; 1.0129x over previous
import functools

import jax
import jax.numpy as jnp
from jax import lax
from jax.experimental import pallas as pl
from jax.experimental.pallas import tpu as pltpu
from jax.experimental.pallas import tpu_sc as plsc

N = 10000
D = 128
C = 16
E = 320000

NC = 2
NS = 16
CH = 128
E_PAD = 327680
CHUNKS = E_PAD // (NS * CH)
NBUF = 4
ROWS_PER_TILE = N // NS
DE1 = 144
W1C = DE1 // NC
W2C = D // NC
BN = 2000
GRID = N // BN


def _make_sc_agg(W):
    mesh = plsc.VectorSubcoreMesh(
        core_axis_name="c", subcore_axis_name="s",
        num_cores=NC, num_subcores=NS)

    @functools.partial(
        pl.kernel,
        out_type=jax.ShapeDtypeStruct((NC, N, W), jnp.float32),
        mesh=mesh,
        scratch_types=[
            pltpu.VMEM((CHUNKS, CH), jnp.int32),
            pltpu.VMEM((CHUNKS, CH), jnp.int32),
            pltpu.VMEM((NBUF, CH, W), jnp.float32),
            pltpu.VMEM_SHARED((N, W), jnp.float32),
            [pltpu.SemaphoreType.DMA] * NBUF,
        ],
        compiler_params=pltpu.CompilerParams(use_tc_tiling_on_sc=False),
    )
    def sc_agg(feats, srcs, dsts, out, src_v, dst_v, ring, acc, gsem):
        c = lax.axis_index("c")
        s = lax.axis_index("s")
        bufs = tuple(ring.at[b] for b in range(NBUF))
        feat = feats.at[c]

        zero16 = jnp.zeros((16,), jnp.float32)

        def zrow(r, carry):
            for k in range(W // 16):
                bufs[0][r, pl.ds(k * 16, 16)] = zero16
            return carry
        lax.fori_loop(0, CH, zrow, 0)
        base = s * ROWS_PER_TILE
        full = ROWS_PER_TILE // CH
        for t in range(full):
            pltpu.sync_copy(bufs[0], acc.at[pl.ds(base + t * CH, CH)])
        rem = ROWS_PER_TILE - full * CH
        pltpu.sync_copy(bufs[0].at[pl.ds(0, rem)],
                        acc.at[pl.ds(base + full * CH, rem)])
        plsc.subcore_barrier()

        pltpu.sync_copy(srcs.at[s], src_v)
        pltpu.sync_copy(dsts.at[s], dst_v)

        for b in range(NBUF):
            pltpu.async_copy(feat.at[src_v.at[b]], bufs[b], gsem[b])

        def body(i, carry):
            j0 = i * NBUF
            for b in range(NBUF):
                pltpu.make_async_copy(feat.at[src_v.at[j0 + b]],
                                      bufs[b], gsem[b]).wait()
                pltpu.sync_copy(bufs[b], acc.at[dst_v.at[j0 + b]], add=True)
                pltpu.async_copy(feat.at[src_v.at[j0 + NBUF + b]],
                                 bufs[b], gsem[b])
            return carry
        lax.fori_loop(0, CHUNKS // NBUF - 1, body, 0)
        j0 = CHUNKS - NBUF
        for b in range(NBUF):
            pltpu.make_async_copy(feat.at[src_v.at[j0 + b]],
                                  bufs[b], gsem[b]).wait()
            pltpu.sync_copy(bufs[b], acc.at[dst_v.at[j0 + b]], add=True)

        plsc.subcore_barrier()
        pltpu.sync_copy(acc.at[pl.ds(base, ROWS_PER_TILE)],
                        out.at[c, pl.ds(base, ROWS_PER_TILE)])

    return sc_agg


_sc_agg = _make_sc_agg(W1C)


def _tc_layer1(x_ref, p0, p1, ws, wn, b, o, oinv):
    pa = p0[...]
    pb = p1[...]
    deg = jnp.sum(pb[:, D - W1C:], axis=1, keepdims=True)
    inv = 1.0 / jnp.maximum(deg, 1.0)
    agg = jnp.concatenate([pa, pb[:, :D - W1C]], axis=1) * inv
    h = x_ref[...] @ ws[...] + agg @ wn[...] + b[...]
    o[...] = jnp.maximum(h, 0.0)
    oinv[...] = jnp.broadcast_to(inv, (BN, 8))


def _tc_layer2(h_ref, q0, q1, dinv, w2s, w2n, b2, wc1, bc1, wc2, bc2, o):
    inv = dinv[...][:, 0:1]
    agg = jnp.concatenate([q0[...], q1[...][:, :D - W1C]], axis=1) * inv
    h2 = jnp.maximum(h_ref[...] @ w2s[...] + agg @ w2n[...] + b2[...], 0.0)
    h3 = jnp.maximum(h2 @ wc1[...] + bc1[...], 0.0)
    o[...] = h3 @ wc2[...] + bc2[...]


def _row_spec(w):
    return pl.BlockSpec((BN, w), lambda i: (i, 0))


def _full_spec(h, w):
    return pl.BlockSpec((h, w), lambda i: (0, 0))


_tck1 = pl.pallas_call(
    _tc_layer1,
    grid=(GRID,),
    in_specs=[_row_spec(D), _row_spec(W1C), _row_spec(W1C),
              _full_spec(D, D), _full_spec(D, D), _full_spec(1, D)],
    out_specs=[_row_spec(D), pl.BlockSpec((BN, 8), lambda i: (i, 0))],
    out_shape=[jax.ShapeDtypeStruct((N, D), jnp.float32),
               jax.ShapeDtypeStruct((N, 8), jnp.float32)],
)

_tck2 = pl.pallas_call(
    _tc_layer2,
    grid=(GRID,),
    in_specs=[_row_spec(D), _row_spec(W1C), _row_spec(W1C),
              pl.BlockSpec((BN, 8), lambda i: (i, 0)),
              _full_spec(D, D), _full_spec(D, D), _full_spec(1, D),
              _full_spec(D, D), _full_spec(1, D),
              _full_spec(D, C), _full_spec(1, C)],
    out_specs=pl.BlockSpec((BN, C), lambda i: (i, 0)),
    out_shape=jax.ShapeDtypeStruct((N, C), jnp.float32),
)


def kernel(x, W1_self, W1_neigh, b1, W2_self, W2_neigh, b2,
           Wc1, bc1, Wc2, bc2, edge_index):
    zrow = jnp.zeros((1, W1C), jnp.float32)
    fa = jnp.concatenate([x[:, :W1C], zrow], axis=0)
    fb = jnp.concatenate([
        jnp.concatenate([x[:, W1C:], jnp.ones((N, 1), jnp.float32),
                         jnp.zeros((N, DE1 - D - 1), jnp.float32)], axis=1),
        zrow], axis=0)
    feats1 = jnp.stack([fa, fb])

    pad = E_PAD - E
    src = jnp.concatenate([edge_index[0], jnp.full((pad,), N, jnp.int32)])
    dst = jnp.concatenate([edge_index[1], jnp.zeros((pad,), jnp.int32)])
    src = src.reshape(NS, CHUNKS, CH)
    dst = dst.reshape(NS, CHUNKS, CH)

    p1 = _sc_agg(feats1, src, dst)
    h1, inv_deg = _tck1(x, p1[0], p1[1], W1_self, W1_neigh, b1.reshape(1, D))

    feats2 = jnp.stack([
        jnp.concatenate([h1[:, :W1C], zrow], axis=0),
        jnp.concatenate([
            jnp.concatenate([h1[:, W1C:],
                             jnp.zeros((N, DE1 - D), jnp.float32)], axis=1),
            zrow], axis=0)])
    p2 = _sc_agg(feats2, src, dst)

    out = _tck2(h1, p2[0], p2[1], inv_deg,
                W2_self, W2_neigh, b2.reshape(1, D),
                Wc1, bc1.reshape(1, D), Wc2, bc2.reshape(1, C))
    return out


# speedup vs baseline: 5.9171x; 1.2127x over previous
import functools

import jax
import jax.numpy as jnp
from jax import lax
from jax.experimental import pallas as pl
from jax.experimental.pallas import tpu as pltpu
from jax.experimental.pallas import tpu_sc as plsc

N = 10000
D = 128
C = 16
E = 320000

NC = 2
NS = 16
CH = 128
E_PAD = 327680
CHUNKS = E_PAD // (NS * CH)
NBUF = 8
ROWS_PER_TILE = N // NS
DE1 = 192
W1C = DE1 // NC
BN = 2000
GRID = N // BN


def _make_sc_agg(W):
    mesh = plsc.VectorSubcoreMesh(
        core_axis_name="c", subcore_axis_name="s",
        num_cores=NC, num_subcores=NS)

    @functools.partial(
        pl.kernel,
        out_type=jax.ShapeDtypeStruct((NC, N, W), jnp.bfloat16),
        mesh=mesh,
        scratch_types=[
            pltpu.VMEM((CHUNKS, CH), jnp.int32),
            pltpu.VMEM((CHUNKS, CH), jnp.int32),
            pltpu.VMEM((NBUF, CH, W), jnp.bfloat16),
            pltpu.VMEM_SHARED((N, W), jnp.bfloat16),
            [pltpu.SemaphoreType.DMA] * NBUF,
        ],
        compiler_params=pltpu.CompilerParams(use_tc_tiling_on_sc=False),
    )
    def sc_agg(feats, srcs, dsts, zinit, out, src_v, dst_v, ring, acc, gsem):
        c = lax.axis_index("c")
        s = lax.axis_index("s")
        bufs = tuple(ring.at[b] for b in range(NBUF))
        feat = feats.at[c]

        base = s * ROWS_PER_TILE
        pltpu.sync_copy(zinit, acc.at[pl.ds(base, ROWS_PER_TILE)])
        plsc.subcore_barrier()

        pltpu.sync_copy(srcs.at[s], src_v)
        pltpu.sync_copy(dsts.at[s], dst_v)

        for b in range(NBUF):
            pltpu.async_copy(feat.at[src_v.at[b]], bufs[b], gsem[b])

        def body(i, carry):
            j0 = i * NBUF
            for b in range(NBUF):
                pltpu.make_async_copy(feat.at[src_v.at[j0 + b]],
                                      bufs[b], gsem[b]).wait()
                pltpu.sync_copy(bufs[b], acc.at[dst_v.at[j0 + b]], add=True)
                pltpu.async_copy(feat.at[src_v.at[j0 + NBUF + b]],
                                 bufs[b], gsem[b])
            return carry
        lax.fori_loop(0, CHUNKS // NBUF - 1, body, 0)
        j0 = CHUNKS - NBUF
        for b in range(NBUF):
            pltpu.make_async_copy(feat.at[src_v.at[j0 + b]],
                                  bufs[b], gsem[b]).wait()
            pltpu.sync_copy(bufs[b], acc.at[dst_v.at[j0 + b]], add=True)

        plsc.subcore_barrier()
        pltpu.sync_copy(acc.at[pl.ds(base, ROWS_PER_TILE)],
                        out.at[c, pl.ds(base, ROWS_PER_TILE)])

    return sc_agg


_sc_agg = _make_sc_agg(W1C)


def _tc_layer1(x_ref, p0, p1, ws, wn, b, o, oinv):
    pa = p0[...].astype(jnp.float32)
    pb = p1[...].astype(jnp.float32)
    deg = jnp.sum(pb[:, D - W1C:], axis=1, keepdims=True)
    inv = 1.0 / jnp.maximum(deg, 1.0)
    agg = jnp.concatenate([pa, pb[:, :D - W1C]], axis=1) * inv
    h = x_ref[...] @ ws[...] + agg @ wn[...] + b[...]
    o[...] = jnp.maximum(h, 0.0)
    oinv[...] = jnp.broadcast_to(inv, (BN, 8))


def _tc_layer2(h_ref, q0, q1, dinv, w2s, w2n, b2, wc1, bc1, wc2, bc2, o):
    inv = dinv[...][:, 0:1]
    agg = jnp.concatenate([q0[...].astype(jnp.float32),
                           q1[...].astype(jnp.float32)[:, :D - W1C]],
                          axis=1) * inv
    h2 = jnp.maximum(h_ref[...] @ w2s[...] + agg @ w2n[...] + b2[...], 0.0)
    h3 = jnp.maximum(h2 @ wc1[...] + bc1[...], 0.0)
    o[...] = h3 @ wc2[...] + bc2[...]


def _row_spec(w):
    return pl.BlockSpec((BN, w), lambda i: (i, 0))


def _full_spec(h, w):
    return pl.BlockSpec((h, w), lambda i: (0, 0))


_tck1 = pl.pallas_call(
    _tc_layer1,
    grid=(GRID,),
    in_specs=[_row_spec(D), _row_spec(W1C), _row_spec(W1C),
              _full_spec(D, D), _full_spec(D, D), _full_spec(1, D)],
    out_specs=[_row_spec(D), pl.BlockSpec((BN, 8), lambda i: (i, 0))],
    out_shape=[jax.ShapeDtypeStruct((N, D), jnp.float32),
               jax.ShapeDtypeStruct((N, 8), jnp.float32)],
)

_tck2 = pl.pallas_call(
    _tc_layer2,
    grid=(GRID,),
    in_specs=[_row_spec(D), _row_spec(W1C), _row_spec(W1C),
              pl.BlockSpec((BN, 8), lambda i: (i, 0)),
              _full_spec(D, D), _full_spec(D, D), _full_spec(1, D),
              _full_spec(D, D), _full_spec(1, D),
              _full_spec(D, C), _full_spec(1, C)],
    out_specs=pl.BlockSpec((BN, C), lambda i: (i, 0)),
    out_shape=jax.ShapeDtypeStruct((N, C), jnp.float32),
)


def kernel(x, W1_self, W1_neigh, b1, W2_self, W2_neigh, b2,
           Wc1, bc1, Wc2, bc2, edge_index):
    zrow = jnp.zeros((1, W1C), jnp.float32)
    fa = jnp.concatenate([x[:, :W1C], zrow], axis=0)
    fb = jnp.concatenate([
        jnp.concatenate([x[:, W1C:], jnp.ones((N, 1), jnp.float32),
                         jnp.zeros((N, DE1 - D - 1), jnp.float32)], axis=1),
        zrow], axis=0)
    feats1 = jnp.stack([fa, fb]).astype(jnp.bfloat16)

    pad = E_PAD - E
    src = jnp.concatenate([edge_index[0], jnp.full((pad,), N, jnp.int32)])
    dst = jnp.concatenate([edge_index[1], jnp.zeros((pad,), jnp.int32)])
    src = src.reshape(NS, CHUNKS, CH)
    dst = dst.reshape(NS, CHUNKS, CH)
    zinit = jnp.zeros((ROWS_PER_TILE, W1C), jnp.bfloat16)

    p1 = _sc_agg(feats1, src, dst, zinit)
    h1, inv_deg = _tck1(x, p1[0], p1[1], W1_self, W1_neigh, b1.reshape(1, D))

    feats2 = jnp.stack([
        jnp.concatenate([h1[:, :W1C], zrow], axis=0),
        jnp.concatenate([
            jnp.concatenate([h1[:, W1C:],
                             jnp.zeros((N, DE1 - D), jnp.float32)], axis=1),
            zrow], axis=0)]).astype(jnp.bfloat16)
    p2 = _sc_agg(feats2, src, dst, zinit)

    out = _tck2(h1, p2[0], p2[1], inv_deg,
                W2_self, W2_neigh, b2.reshape(1, D),
                Wc1, bc1.reshape(1, D), Wc2, bc2.reshape(1, C))
    return out
